# baseline (device time: 13079 ns/iter reference)
import jax
import jax.numpy as jnp
from jax import lax
from jax.experimental import pallas as pl
from jax.experimental.pallas import tpu as pltpu

C = 8


def kernel(ids, E):
    v_per, d = E.shape
    t = ids.shape[0]
    half = t // 2
    rows = half // C

    my_x = lax.axis_index("x")
    my_y = lax.axis_index("y")

    my_ids = lax.dynamic_slice(ids, (my_x * half,), (half,))
    local = my_ids - my_y * v_per
    gathered = E[local]
    mine = (local >= 0) & (local < v_per)
    sel = mine[:, None].astype(jnp.bfloat16)

    def body(g_ref, m_ref, out_ref, p_ref, red_ref, ycomm_ref,
             ysend, yrecv, xsend, xrecv, dsem):
        my_x = lax.axis_index("x")
        my_y = lax.axis_index("y")
        ypeer = (my_x, 1 - my_y)
        xpeer = (1 - my_x, my_y)

        barrier = pltpu.get_barrier_semaphore()
        for peer in (ypeer, xpeer):
            pl.semaphore_signal(
                barrier, inc=1, device_id=peer,
                device_id_type=pl.DeviceIdType.MESH,
            )
        p_ref[...] = g_ref[...].astype(jnp.bfloat16)
        pl.semaphore_wait(barrier, 2)

        yr = []
        for c in range(C):
            sl = pl.ds(c * rows, rows)
            r = pltpu.make_async_remote_copy(
                src_ref=p_ref.at[sl],
                dst_ref=ycomm_ref.at[sl],
                send_sem=ysend.at[c],
                recv_sem=yrecv.at[c],
                device_id=ypeer,
                device_id_type=pl.DeviceIdType.MESH,
            )
            r.start()
            yr.append(r)

        my_off = my_x * half

        xr = []
        cps = []
        for c in range(C):
            sl = pl.ds(c * rows, rows)
            osl = pl.ds(my_off + c * rows, rows)
            yr[c].wait_recv()
            red_ref[sl, :] = jnp.where(
                m_ref[sl, :] != 0, p_ref[sl, :], ycomm_ref[sl, :]
            )
            r = pltpu.make_async_remote_copy(
                src_ref=red_ref.at[sl],
                dst_ref=out_ref.at[osl],
                send_sem=xsend.at[c],
                recv_sem=xrecv.at[c],
                device_id=xpeer,
                device_id_type=pl.DeviceIdType.MESH,
            )
            r.start()
            xr.append(r)
            cp = pltpu.make_async_copy(red_ref.at[sl], out_ref.at[osl],
                                       dsem.at[c])
            cp.start()
            cps.append(cp)

        for c in range(C):
            xr[c].wait_recv()
        for cp in cps:
            cp.wait()
        for r in yr:
            r.wait_send()
        for r in xr:
            r.wait_send()

    return pl.pallas_call(
        body,
        out_shape=jax.ShapeDtypeStruct((t, d), jnp.bfloat16),
        in_specs=[
            pl.BlockSpec(memory_space=pltpu.VMEM),
            pl.BlockSpec(memory_space=pltpu.VMEM),
        ],
        out_specs=pl.BlockSpec(memory_space=pltpu.MemorySpace.HBM),
        scratch_shapes=[
            pltpu.VMEM((half, d), jnp.bfloat16),
            pltpu.VMEM((half, d), jnp.bfloat16),
            pltpu.VMEM((half, d), jnp.bfloat16),
            pltpu.SemaphoreType.DMA((C,)),
            pltpu.SemaphoreType.DMA((C,)),
            pltpu.SemaphoreType.DMA((C,)),
            pltpu.SemaphoreType.DMA((C,)),
            pltpu.SemaphoreType.DMA((C,)),
        ],
        compiler_params=pltpu.CompilerParams(collective_id=0),
    )(gathered, sel)


# device time: 12272 ns/iter; 1.0658x vs baseline; 1.0658x over previous
import jax
import jax.numpy as jnp
from jax import lax
from jax.experimental import pallas as pl
from jax.experimental.pallas import tpu as pltpu

C = 8


def kernel(ids, E):
    v_per, d = E.shape
    t = ids.shape[0]
    half = t // 2
    rows = half // C

    my_x = lax.axis_index("x")
    my_y = lax.axis_index("y")

    my_ids = lax.dynamic_slice(ids, (my_x * half,), (half,))
    local = my_ids - my_y * v_per
    mask = (local >= 0) & (local < v_per)
    safe = jnp.where(mask, local, 0)
    gathered = jnp.where(mask[:, None], E[safe], 0.0)

    def body(g_ref, out_ref, p_ref, red_ref, ycomm_ref,
             ysend, yrecv, xsend, xrecv, dsem):
        my_x = lax.axis_index("x")
        my_y = lax.axis_index("y")
        ypeer = (my_x, 1 - my_y)
        xpeer = (1 - my_x, my_y)

        barrier = pltpu.get_barrier_semaphore()
        for peer in (ypeer, xpeer):
            pl.semaphore_signal(
                barrier, inc=1, device_id=peer,
                device_id_type=pl.DeviceIdType.MESH,
            )
        p_ref[...] = g_ref[...].astype(jnp.bfloat16)
        pl.semaphore_wait(barrier, 2)

        yr = []
        for c in range(C):
            sl = pl.ds(c * rows, rows)
            r = pltpu.make_async_remote_copy(
                src_ref=p_ref.at[sl],
                dst_ref=ycomm_ref.at[sl],
                send_sem=ysend.at[c],
                recv_sem=yrecv.at[c],
                device_id=ypeer,
                device_id_type=pl.DeviceIdType.MESH,
            )
            r.start()
            yr.append(r)

        my_off = my_x * half

        xr = []
        cps = []
        for c in range(C):
            sl = pl.ds(c * rows, rows)
            osl = pl.ds(my_off + c * rows, rows)
            yr[c].wait_recv()
            red_ref[sl, :] = p_ref[sl, :] + ycomm_ref[sl, :]
            r = pltpu.make_async_remote_copy(
                src_ref=red_ref.at[sl],
                dst_ref=out_ref.at[osl],
                send_sem=xsend.at[c],
                recv_sem=xrecv.at[c],
                device_id=xpeer,
                device_id_type=pl.DeviceIdType.MESH,
            )
            r.start()
            xr.append(r)
            cp = pltpu.make_async_copy(red_ref.at[sl], out_ref.at[osl],
                                       dsem.at[c])
            cp.start()
            cps.append(cp)

        for c in range(C):
            xr[c].wait_recv()
        for cp in cps:
            cp.wait()
        for r in yr:
            r.wait_send()
        for r in xr:
            r.wait_send()

    return pl.pallas_call(
        body,
        out_shape=jax.ShapeDtypeStruct((t, d), jnp.bfloat16),
        in_specs=[pl.BlockSpec(memory_space=pltpu.VMEM)],
        out_specs=pl.BlockSpec(memory_space=pltpu.MemorySpace.HBM),
        scratch_shapes=[
            pltpu.VMEM((half, d), jnp.bfloat16),
            pltpu.VMEM((half, d), jnp.bfloat16),
            pltpu.VMEM((half, d), jnp.bfloat16),
            pltpu.SemaphoreType.DMA((C,)),
            pltpu.SemaphoreType.DMA((C,)),
            pltpu.SemaphoreType.DMA((C,)),
            pltpu.SemaphoreType.DMA((C,)),
            pltpu.SemaphoreType.DMA((C,)),
        ],
        compiler_params=pltpu.CompilerParams(collective_id=0),
    )(gathered)
